# baseline (speedup 1.0000x reference)
import functools

import jax
import jax.numpy as jnp
from jax import lax
from jax.experimental import pallas as pl
from jax.experimental.pallas import tpu as pltpu
from jax.experimental.pallas import tpu_sc as plsc

N = 10000
E = 320000
D = 128
G = 16

NTILES = 32
CPT = D // NTILES
FLAT = N * CPT
ECH = 8000
NCH = E // ECH
RB = 1000
NRB = N // RB


def _mm1_body(x_ref, w_ref, b_ref, u_ref, v_ref):
    x = x_ref[...]
    w = w_ref[...]
    wb = w[D:, :]
    v_ref[...] = jnp.dot(x, wb, preferred_element_type=jnp.float32)
    u_ref[...] = (
        jnp.dot(x, w[:D, :] - wb, preferred_element_type=jnp.float32)
        + b_ref[...]
    )


def _mm2_body(u1_ref, m1_ref, w_ref, b_ref, u_ref, v_ref):
    m = m1_ref[...]
    h = jnp.where(jnp.isfinite(m), u1_ref[...] + m, 0.0)
    w = w_ref[...]
    wb = w[D:, :]
    v_ref[...] = jnp.dot(h, wb, preferred_element_type=jnp.float32)
    u_ref[...] = (
        jnp.dot(h, w[:D, :] - wb, preferred_element_type=jnp.float32)
        + b_ref[...]
    )


def _pool_body(u_ref, m_ref, batch_ref, out_ref):
    i = pl.program_id(0)

    @pl.when(i == 0)
    def _():
        out_ref[...] = jnp.full((G, D), -jnp.inf, dtype=jnp.float32)

    m = m_ref[...]
    h2 = jnp.where(jnp.isfinite(m), u_ref[...] + m, 0.0)
    b = batch_ref[...]
    parts = jnp.concatenate(
        [
            jnp.max(jnp.where(b == g, h2, -jnp.inf), axis=0, keepdims=True)
            for g in range(G)
        ],
        axis=0,
    )
    out_ref[...] = jnp.maximum(out_ref[...], parts)

    @pl.when(i == NRB - 1)
    def _():
        o = out_ref[...]
        out_ref[...] = jnp.where(jnp.isfinite(o), o, 0.0)


def _mm1(x, w, b):
    return pl.pallas_call(
        _mm1_body,
        grid=(NRB,),
        in_specs=[
            pl.BlockSpec((RB, D), lambda i: (i, 0)),
            pl.BlockSpec((2 * D, D), lambda i: (0, 0)),
            pl.BlockSpec((1, D), lambda i: (0, 0)),
        ],
        out_specs=[
            pl.BlockSpec((RB, D), lambda i: (i, 0)),
            pl.BlockSpec((RB, D), lambda i: (i, 0)),
        ],
        out_shape=[
            jax.ShapeDtypeStruct((N, D), jnp.float32),
            jax.ShapeDtypeStruct((N, D), jnp.float32),
        ],
    )(x, w, b)


def _mm2(u1, m1, w, b):
    return pl.pallas_call(
        _mm2_body,
        grid=(NRB,),
        in_specs=[
            pl.BlockSpec((RB, D), lambda i: (i, 0)),
            pl.BlockSpec((RB, D), lambda i: (i, 0)),
            pl.BlockSpec((2 * D, D), lambda i: (0, 0)),
            pl.BlockSpec((1, D), lambda i: (0, 0)),
        ],
        out_specs=[
            pl.BlockSpec((RB, D), lambda i: (i, 0)),
            pl.BlockSpec((RB, D), lambda i: (i, 0)),
        ],
        out_shape=[
            jax.ShapeDtypeStruct((N, D), jnp.float32),
            jax.ShapeDtypeStruct((N, D), jnp.float32),
        ],
    )(u1, m1, w, b)


def _pool(u2, m2, batch2d):
    return pl.pallas_call(
        _pool_body,
        grid=(NRB,),
        in_specs=[
            pl.BlockSpec((RB, D), lambda i: (i, 0)),
            pl.BlockSpec((RB, D), lambda i: (i, 0)),
            pl.BlockSpec((RB, 1), lambda i: (i, 0)),
        ],
        out_specs=pl.BlockSpec((G, D), lambda i: (0, 0)),
        out_shape=jax.ShapeDtypeStruct((G, D), jnp.float32),
    )(u2, m2, batch2d)


def _segmax(vb, src, dst):
    mesh = plsc.VectorSubcoreMesh(core_axis_name="c", subcore_axis_name="s")

    @functools.partial(
        pl.kernel,
        out_type=jax.ShapeDtypeStruct((NTILES, FLAT), jnp.float32),
        mesh=mesh,
        compiler_params=pltpu.CompilerParams(needs_layout_passes=False),
        scratch_types=[
            pltpu.VMEM((FLAT,), jnp.float32),
            pltpu.VMEM((FLAT,), jnp.float32),
            pltpu.VMEM((ECH,), jnp.int32),
            pltpu.VMEM((ECH,), jnp.int32),
        ],
    )
    def k(vb_hbm, src_hbm, dst_hbm, out_hbm, vloc, mloc, sbuf, dbuf):
        wid = lax.axis_index("c") * 16 + lax.axis_index("s")
        pltpu.sync_copy(vb_hbm.at[wid], vloc)

        neg_inf = jnp.broadcast_to(jnp.float32(-jnp.inf), (16,))

        def init_body(i, carry):
            mloc[pl.ds(i * 16, 16)] = neg_inf
            return carry

        lax.fori_loop(0, FLAT // 16, init_body, 0)

        lanes = lax.iota(jnp.int32, 16)
        lane_col = jnp.minimum(lanes, CPT - 1)
        lane_mask = lanes < CPT

        def chunk_body(ci, carry):
            pltpu.sync_copy(src_hbm.at[pl.ds(ci * ECH, ECH)], sbuf)
            pltpu.sync_copy(dst_hbm.at[pl.ds(ci * ECH, ECH)], dbuf)

            def edge_body(e, c2):
                e_b = jnp.broadcast_to(e, (16,))
                s = plsc.load_gather(sbuf, [e_b])
                d = plsc.load_gather(dbuf, [e_b])
                vidx = s * CPT + lane_col
                midx = d * CPT + lane_col
                v = plsc.load_gather(vloc, [vidx], mask=lane_mask)
                m = plsc.load_gather(mloc, [midx], mask=lane_mask)
                plsc.store_scatter(
                    mloc, [midx], jnp.maximum(v, m), mask=lane_mask
                )
                return c2

            lax.fori_loop(0, ECH, edge_body, 0)
            return carry

        lax.fori_loop(0, NCH, chunk_body, 0)
        pltpu.sync_copy(mloc, out_hbm.at[wid])

    return k(vb, src, dst)


def _to_blocked(v):
    return v.reshape(N, NTILES, CPT).transpose(1, 0, 2).reshape(NTILES, FLAT)


def _from_blocked(mb):
    return mb.reshape(NTILES, N, CPT).transpose(1, 0, 2).reshape(N, D)


def kernel(x, edge_index, batch, W1, b1, W2, b2):
    src = edge_index[0]
    dst = edge_index[1]
    b1r = b1.reshape(1, D)
    b2r = b2.reshape(1, D)

    u1, v1 = _mm1(x, W1, b1r)
    m1 = _from_blocked(_segmax(_to_blocked(v1), src, dst))
    u2, v2 = _mm2(u1, m1, W2, b2r)
    m2 = _from_blocked(_segmax(_to_blocked(v2), src, dst))
    return _pool(u2, m2, batch.reshape(N, 1))


# speedup vs baseline: 2.7369x; 2.7369x over previous
import functools

import jax
import jax.numpy as jnp
from jax import lax
from jax.experimental import pallas as pl
from jax.experimental.pallas import tpu as pltpu
from jax.experimental.pallas import tpu_sc as plsc

N = 10000
E = 320000
D = 128
G = 16

NTILES = 32
CPT = D // NTILES
FLAT = N * CPT
ECH = 8000
NCH = E // ECH
RB = 1000
NRB = N // RB


def _mm1_body(x_ref, w_ref, b_ref, u_ref, v_ref):
    x = x_ref[...]
    w = w_ref[...]
    wb = w[D:, :]
    v_ref[...] = jnp.dot(x, wb, preferred_element_type=jnp.float32)
    u_ref[...] = (
        jnp.dot(x, w[:D, :] - wb, preferred_element_type=jnp.float32)
        + b_ref[...]
    )


def _mm2_body(u1_ref, m1_ref, w_ref, b_ref, u_ref, v_ref):
    m = m1_ref[...]
    h = jnp.where(jnp.isfinite(m), u1_ref[...] + m, 0.0)
    w = w_ref[...]
    wb = w[D:, :]
    v_ref[...] = jnp.dot(h, wb, preferred_element_type=jnp.float32)
    u_ref[...] = (
        jnp.dot(h, w[:D, :] - wb, preferred_element_type=jnp.float32)
        + b_ref[...]
    )


def _pool_body(u_ref, m_ref, batch_ref, out_ref):
    i = pl.program_id(0)

    @pl.when(i == 0)
    def _():
        out_ref[...] = jnp.full((G, D), -jnp.inf, dtype=jnp.float32)

    m = m_ref[...]
    h2 = jnp.where(jnp.isfinite(m), u_ref[...] + m, 0.0)
    b = batch_ref[...]
    parts = jnp.concatenate(
        [
            jnp.max(jnp.where(b == g, h2, -jnp.inf), axis=0, keepdims=True)
            for g in range(G)
        ],
        axis=0,
    )
    out_ref[...] = jnp.maximum(out_ref[...], parts)

    @pl.when(i == NRB - 1)
    def _():
        o = out_ref[...]
        out_ref[...] = jnp.where(jnp.isfinite(o), o, 0.0)


def _mm1(x, w, b):
    return pl.pallas_call(
        _mm1_body,
        grid=(NRB,),
        in_specs=[
            pl.BlockSpec((RB, D), lambda i: (i, 0)),
            pl.BlockSpec((2 * D, D), lambda i: (0, 0)),
            pl.BlockSpec((1, D), lambda i: (0, 0)),
        ],
        out_specs=[
            pl.BlockSpec((RB, D), lambda i: (i, 0)),
            pl.BlockSpec((RB, D), lambda i: (i, 0)),
        ],
        out_shape=[
            jax.ShapeDtypeStruct((N, D), jnp.float32),
            jax.ShapeDtypeStruct((N, D), jnp.float32),
        ],
    )(x, w, b)


def _mm2(u1, m1, w, b):
    return pl.pallas_call(
        _mm2_body,
        grid=(NRB,),
        in_specs=[
            pl.BlockSpec((RB, D), lambda i: (i, 0)),
            pl.BlockSpec((RB, D), lambda i: (i, 0)),
            pl.BlockSpec((2 * D, D), lambda i: (0, 0)),
            pl.BlockSpec((1, D), lambda i: (0, 0)),
        ],
        out_specs=[
            pl.BlockSpec((RB, D), lambda i: (i, 0)),
            pl.BlockSpec((RB, D), lambda i: (i, 0)),
        ],
        out_shape=[
            jax.ShapeDtypeStruct((N, D), jnp.float32),
            jax.ShapeDtypeStruct((N, D), jnp.float32),
        ],
    )(u1, m1, w, b)


def _pool(u2, m2, batch2d):
    return pl.pallas_call(
        _pool_body,
        grid=(NRB,),
        in_specs=[
            pl.BlockSpec((RB, D), lambda i: (i, 0)),
            pl.BlockSpec((RB, D), lambda i: (i, 0)),
            pl.BlockSpec((RB, 1), lambda i: (i, 0)),
        ],
        out_specs=pl.BlockSpec((G, D), lambda i: (0, 0)),
        out_shape=jax.ShapeDtypeStruct((G, D), jnp.float32),
    )(u2, m2, batch2d)


def _segmax(vb, src, dst):
    mesh = plsc.VectorSubcoreMesh(core_axis_name="c", subcore_axis_name="s")

    @functools.partial(
        pl.kernel,
        out_type=jax.ShapeDtypeStruct((NTILES, FLAT), jnp.float32),
        mesh=mesh,
        compiler_params=pltpu.CompilerParams(needs_layout_passes=False),
        scratch_types=[
            pltpu.VMEM((FLAT,), jnp.float32),
            pltpu.VMEM((FLAT,), jnp.float32),
            pltpu.VMEM((ECH,), jnp.int32),
            pltpu.VMEM((ECH,), jnp.int32),
            pltpu.VMEM((N,), jnp.int32),
        ],
    )
    def k(vb_hbm, src_hbm, dst_hbm, out_hbm, vloc, mloc, sbuf, dbuf, tmp):
        wid = lax.axis_index("c") * 16 + lax.axis_index("s")
        pltpu.sync_copy(vb_hbm.at[wid], vloc)

        neg_inf = jnp.broadcast_to(jnp.float32(-jnp.inf), (16,))

        def init_body(i, carry):
            mloc[pl.ds(i * 16, 16)] = neg_inf
            return carry

        lax.fori_loop(0, FLAT // 16, init_body, 0)

        lanes = lax.iota(jnp.int32, 16)
        lane_col = jnp.minimum(lanes, CPT - 1)
        lane_mask = lanes < CPT

        def chunk_body(ci, carry):
            pltpu.sync_copy(src_hbm.at[pl.ds(ci * ECH, ECH)], sbuf)
            pltpu.sync_copy(dst_hbm.at[pl.ds(ci * ECH, ECH)], dbuf)

            def group_body(g, c2):
                s = sbuf[pl.ds(g * 16, 16)]
                d = dbuf[pl.ds(g * 16, 16)]
                plsc.store_scatter(tmp, [d], lanes)
                rb = plsc.load_gather(tmp, [d])
                conflict = jnp.any(rb != lanes)

                def fast():
                    s4 = s * CPT
                    d4 = d * CPT
                    for c in range(CPT):
                        v = plsc.load_gather(vloc, [s4 + c])
                        m = plsc.load_gather(mloc, [d4 + c])
                        plsc.store_scatter(mloc, [d4 + c], jnp.maximum(v, m))

                def slow():
                    def edge_body(j, c3):
                        e_b = jnp.broadcast_to(g * 16 + j, (16,))
                        sv = plsc.load_gather(sbuf, [e_b])
                        dv = plsc.load_gather(dbuf, [e_b])
                        vidx = sv * CPT + lane_col
                        midx = dv * CPT + lane_col
                        v = plsc.load_gather(vloc, [vidx], mask=lane_mask)
                        m = plsc.load_gather(mloc, [midx], mask=lane_mask)
                        plsc.store_scatter(
                            mloc, [midx], jnp.maximum(v, m), mask=lane_mask
                        )
                        return c3

                    lax.fori_loop(0, 16, edge_body, 0)

                lax.cond(conflict, slow, fast)
                return c2

            lax.fori_loop(0, ECH // 16, group_body, 0)
            return carry

        lax.fori_loop(0, NCH, chunk_body, 0)
        pltpu.sync_copy(mloc, out_hbm.at[wid])

    return k(vb, src, dst)


def _to_blocked(v):
    return v.reshape(N, NTILES, CPT).transpose(1, 0, 2).reshape(NTILES, FLAT)


def _from_blocked(mb):
    return mb.reshape(NTILES, N, CPT).transpose(1, 0, 2).reshape(N, D)


def kernel(x, edge_index, batch, W1, b1, W2, b2):
    src = edge_index[0]
    dst = edge_index[1]
    b1r = b1.reshape(1, D)
    b2r = b2.reshape(1, D)

    u1, v1 = _mm1(x, W1, b1r)
    m1 = _from_blocked(_segmax(_to_blocked(v1), src, dst))
    u2, v2 = _mm2(u1, m1, W2, b2r)
    m2 = _from_blocked(_segmax(_to_blocked(v2), src, dst))
    return _pool(u2, m2, batch.reshape(N, 1))
